# baseline (speedup 1.0000x reference)
import functools

import jax
import jax.numpy as jnp
from jax import lax
from jax.experimental import pallas as pl
from jax.experimental.pallas import tpu as pltpu
from jax.experimental.pallas import tpu_sc as plsc

N_NODES = 10000
D_EDGE = 16
HIDDEN = 16
MSG = 16

_SC_PARAMS = pltpu.CompilerParams(use_tc_tiling_on_sc=False)

NC, NS = 2, 16
NW = NC * NS
ROW = 128
CHUNK_ROWS = 40
CHUNK_EDGES = CHUNK_ROWS * ROW
PACK = 128 // HIDDEN
TILE_W = 320


def _make_gather(ep):
    rows_per_worker = (ep // ROW) // NW
    n_chunks = rows_per_worker // CHUNK_ROWS
    mesh = plsc.VectorSubcoreMesh(core_axis_name="c", subcore_axis_name="s")

    @functools.partial(
        pl.kernel,
        out_type=jax.ShapeDtypeStruct((ep, HIDDEN), jnp.float32),
        mesh=mesh,
        scratch_types=[
            pltpu.VMEM((CHUNK_ROWS, ROW), jnp.int32),
            pltpu.VMEM((CHUNK_EDGES, HIDDEN), jnp.float32),
            pltpu.VMEM_SHARED((N_NODES, HIDDEN), jnp.float32),
            pltpu.SemaphoreType.DMA,
        ],
        compiler_params=_SC_PARAMS,
    )
    def gather(hidden_hbm, src_hbm, neigh_hbm, idx_v, rows_v, hid_sh, sem):
        cid = lax.axis_index("c")
        sid = lax.axis_index("s")
        wid = sid * NC + cid

        @pl.when(sid == 0)
        def _():
            pltpu.sync_copy(hidden_hbm, hid_sh)

        plsc.subcore_barrier()

        @pl.loop(0, n_chunks)
        def _(ci):
            row0 = wid * rows_per_worker + ci * CHUNK_ROWS

            pltpu.sync_copy(src_hbm.at[pl.ds(row0, CHUNK_ROWS)], idx_v)

            @pl.loop(0, CHUNK_ROWS)
            def _(j):
                pltpu.async_copy(
                    hid_sh.at[idx_v.at[j]],
                    rows_v.at[pl.ds(j * ROW, ROW)],
                    sem,
                )

            pltpu.make_async_copy(
                neigh_hbm.at[pl.ds(0, CHUNK_EDGES)], rows_v, sem
            ).wait()

            pltpu.sync_copy(rows_v, neigh_hbm.at[pl.ds(row0 * ROW, CHUNK_EDGES)])

    return gather


def _make_scatter(ep):
    rows_per_worker = (ep // ROW) // NW
    n_chunks = rows_per_worker // CHUNK_ROWS
    mesh = plsc.VectorSubcoreMesh(core_axis_name="c", subcore_axis_name="s")

    @functools.partial(
        pl.kernel,
        out_type=jax.ShapeDtypeStruct((NC, N_NODES, MSG), jnp.float32),
        mesh=mesh,
        scratch_types=[
            pltpu.VMEM((CHUNK_ROWS, ROW), jnp.int32),
            pltpu.VMEM((CHUNK_EDGES, MSG), jnp.float32),
            pltpu.VMEM_SHARED((N_NODES, MSG), jnp.float32),
            pltpu.SemaphoreType.DMA,
        ],
        compiler_params=_SC_PARAMS,
    )
    def scatter(msg_hbm, tgt_hbm, zero_hbm, out_hbm, idx_v, rows_v, acc_sh, sem):
        cid = lax.axis_index("c")
        sid = lax.axis_index("s")

        @pl.when(sid == 0)
        def _():
            pltpu.sync_copy(zero_hbm, acc_sh)

        plsc.subcore_barrier()

        @pl.loop(0, n_chunks)
        def _(ci):
            row0 = (cid * NS + sid) * rows_per_worker + ci * CHUNK_ROWS

            pltpu.sync_copy(tgt_hbm.at[pl.ds(row0, CHUNK_ROWS)], idx_v)
            pltpu.sync_copy(msg_hbm.at[pl.ds(row0 * ROW, CHUNK_EDGES)], rows_v)

            @pl.loop(0, CHUNK_ROWS)
            def _(j):
                pltpu.sync_copy(
                    rows_v.at[pl.ds(j * ROW, ROW)],
                    acc_sh.at[idx_v.at[j]],
                    add=True,
                )

        plsc.subcore_barrier()

        @pl.when(sid == 0)
        def _():
            pltpu.sync_copy(acc_sh, out_hbm.at[cid])

    return scatter


def _dot(a, b):
    return lax.dot_general(a, b, (((1,), (0,)), ((), ())),
                           preferred_element_type=jnp.float32)


def _mm_body(n_real_wrows, ef_ref, ng_ref, r_ref, s_ref, wr_ref, br_ref,
             out_ref):
    ef = ef_ref[...].astype(jnp.bfloat16)
    ng = ng_ref[...].astype(jnp.bfloat16)
    z = (_dot(ef, r_ref[...]) * _dot(ng, s_ref[...])).astype(jnp.bfloat16)
    m = _dot(z, wr_ref[...]) + _dot(ng, br_ref[...])
    row0 = pl.program_id(0) * TILE_W
    rid = row0 + lax.broadcasted_iota(jnp.int32, (TILE_W, 128), 0)
    out_ref[...] = jnp.where(rid < n_real_wrows, m, 0.0)


def _add_body(p_ref, o_ref):
    o_ref[...] = p_ref[0] + p_ref[1]


def _pad_body(n_real_rows, n_pad_rows, s_ref, t_ref, sp_ref, tp_ref):
    zero_tail = jnp.zeros((n_pad_rows, ROW), jnp.int32)
    sp_ref[pl.ds(0, n_real_rows), :] = s_ref[...]
    sp_ref[pl.ds(n_real_rows, n_pad_rows), :] = zero_tail
    tp_ref[pl.ds(0, n_real_rows), :] = t_ref[...]
    tp_ref[pl.ds(n_real_rows, n_pad_rows), :] = zero_tail


def kernel(node_features, edge_features, edge_sources, edge_targets, hidden,
           initial, W, b):
    E = edge_features.shape[0]
    granule = NW * CHUNK_ROWS * ROW
    ep = ((E + granule - 1) // granule) * granule

    n_real_rows = E // ROW
    n_pad_rows = ep // ROW - n_real_rows
    src_r = edge_sources.reshape(n_real_rows, ROW)
    tgt_r = edge_targets.reshape(n_real_rows, ROW)
    src_p, tgt_p = pl.pallas_call(
        functools.partial(_pad_body, n_real_rows, n_pad_rows),
        out_shape=(jax.ShapeDtypeStruct((ep // ROW, ROW), jnp.int32),
                   jax.ShapeDtypeStruct((ep // ROW, ROW), jnp.int32)),
    )(src_r, tgt_r)

    neigh = _make_gather(ep)(hidden, src_p)

    Wr = W.reshape(D_EDGE, MSG, HIDDEN).transpose(0, 2, 1).reshape(
        D_EDGE * HIDDEN, MSG).astype(jnp.bfloat16)
    Br = b.reshape(MSG, HIDDEN).T.astype(jnp.bfloat16)
    R = jnp.repeat(jnp.eye(D_EDGE, dtype=jnp.bfloat16), HIDDEN, axis=1)
    S = jnp.tile(jnp.eye(HIDDEN, dtype=jnp.bfloat16), (1, D_EDGE))
    eye8 = jnp.eye(PACK, dtype=jnp.bfloat16)
    Rb = jnp.kron(eye8, R)
    Sb = jnp.kron(eye8, S)
    Wrb = jnp.kron(eye8, Wr)
    Brb = jnp.kron(eye8, Br)

    ef_w = edge_features.reshape(E // PACK, 128)
    neigh_w = neigh.reshape(ep // PACK, 128)
    n_tiles = ep // PACK // TILE_W
    n_real_wrows = E // PACK
    clamp = n_real_wrows // TILE_W - 1
    messages = pl.pallas_call(
        functools.partial(_mm_body, n_real_wrows),
        grid=(n_tiles,),
        in_specs=[
            pl.BlockSpec((TILE_W, 128), lambda i: (jnp.minimum(i, clamp), 0)),
            pl.BlockSpec((TILE_W, 128), lambda i: (i, 0)),
            pl.BlockSpec((128, PACK * D_EDGE * HIDDEN), lambda i: (0, 0)),
            pl.BlockSpec((128, PACK * D_EDGE * HIDDEN), lambda i: (0, 0)),
            pl.BlockSpec((PACK * D_EDGE * HIDDEN, 128), lambda i: (0, 0)),
            pl.BlockSpec((128, 128), lambda i: (0, 0)),
        ],
        out_specs=pl.BlockSpec((TILE_W, 128), lambda i: (i, 0)),
        out_shape=jax.ShapeDtypeStruct((ep // PACK, 128), jnp.float32),
    )(ef_w, neigh_w, Rb, Sb, Wrb, Brb)

    zeros = jnp.zeros((N_NODES, MSG), jnp.float32)
    partials = _make_scatter(ep)(messages.reshape(ep, MSG), tgt_p, zeros)

    out = pl.pallas_call(
        _add_body,
        out_shape=jax.ShapeDtypeStruct((N_NODES, MSG), jnp.float32),
    )(partials)
    return out


# speedup vs baseline: 1.0215x; 1.0215x over previous
import functools

import jax
import jax.numpy as jnp
from jax import lax
from jax.experimental import pallas as pl
from jax.experimental.pallas import tpu as pltpu
from jax.experimental.pallas import tpu_sc as plsc

N_NODES = 10000
D_EDGE = 16
HIDDEN = 16
MSG = 16

_SC_PARAMS = pltpu.CompilerParams(use_tc_tiling_on_sc=False)

NC, NS = 2, 16
NW = NC * NS
ROW = 128
CHUNK_ROWS = 40
CHUNK_EDGES = CHUNK_ROWS * ROW
PACK = 128 // HIDDEN
TILE_W = 320


def _make_gather(ep):
    rows_per_worker = (ep // ROW) // NW
    n_chunks = rows_per_worker // CHUNK_ROWS
    mesh = plsc.VectorSubcoreMesh(core_axis_name="c", subcore_axis_name="s")

    @functools.partial(
        pl.kernel,
        out_type=jax.ShapeDtypeStruct((ep, HIDDEN), jnp.float32),
        mesh=mesh,
        scratch_types=[
            pltpu.VMEM((CHUNK_ROWS, ROW), jnp.int32),
            pltpu.VMEM((CHUNK_EDGES, HIDDEN), jnp.float32),
            pltpu.VMEM_SHARED((N_NODES, HIDDEN), jnp.float32),
            pltpu.SemaphoreType.DMA,
        ],
        compiler_params=_SC_PARAMS,
    )
    def gather(hidden_hbm, src_hbm, neigh_hbm, idx_v, rows_v, hid_sh, sem):
        cid = lax.axis_index("c")
        sid = lax.axis_index("s")
        wid = sid * NC + cid

        @pl.when(sid == 0)
        def _():
            pltpu.sync_copy(hidden_hbm, hid_sh)

        plsc.subcore_barrier()

        @pl.loop(0, n_chunks)
        def _(ci):
            row0 = wid * rows_per_worker + ci * CHUNK_ROWS

            pltpu.sync_copy(src_hbm.at[pl.ds(row0, CHUNK_ROWS)], idx_v)

            @pl.loop(0, CHUNK_ROWS)
            def _(j):
                pltpu.async_copy(
                    hid_sh.at[idx_v.at[j]],
                    rows_v.at[pl.ds(j * ROW, ROW)],
                    sem,
                )

            pltpu.make_async_copy(
                neigh_hbm.at[pl.ds(0, CHUNK_EDGES)], rows_v, sem
            ).wait()

            pltpu.sync_copy(rows_v, neigh_hbm.at[pl.ds(row0 * ROW, CHUNK_EDGES)])

    return gather


def _make_scatter(ep):
    rows_per_worker = (ep // ROW) // NW
    n_chunks = rows_per_worker // CHUNK_ROWS
    mesh = plsc.VectorSubcoreMesh(core_axis_name="c", subcore_axis_name="s")

    @functools.partial(
        pl.kernel,
        out_type=jax.ShapeDtypeStruct((NC, N_NODES, MSG), jnp.float32),
        mesh=mesh,
        scratch_types=[
            pltpu.VMEM((CHUNK_ROWS, ROW), jnp.int32),
            pltpu.VMEM((CHUNK_EDGES, MSG), jnp.float32),
            pltpu.VMEM_SHARED((N_NODES, MSG), jnp.float32),
            pltpu.SemaphoreType.DMA,
        ],
        compiler_params=_SC_PARAMS,
    )
    def scatter(msg_hbm, tgt_hbm, zero_hbm, out_hbm, idx_v, rows_v, acc_sh, sem):
        cid = lax.axis_index("c")
        sid = lax.axis_index("s")

        @pl.when(sid == 0)
        def _():
            pltpu.sync_copy(zero_hbm, acc_sh)

        plsc.subcore_barrier()

        @pl.loop(0, n_chunks)
        def _(ci):
            row0 = (cid * NS + sid) * rows_per_worker + ci * CHUNK_ROWS

            pltpu.sync_copy(tgt_hbm.at[pl.ds(row0, CHUNK_ROWS)], idx_v)
            pltpu.sync_copy(msg_hbm.at[pl.ds(row0 * ROW, CHUNK_EDGES)], rows_v)

            @pl.loop(0, CHUNK_ROWS)
            def _(j):
                pltpu.sync_copy(
                    rows_v.at[pl.ds(j * ROW, ROW)],
                    acc_sh.at[idx_v.at[j]],
                    add=True,
                )

        plsc.subcore_barrier()

        @pl.when(sid == 0)
        def _():
            pltpu.sync_copy(acc_sh, out_hbm.at[cid])

    return scatter


def _dot(a, b, out_dtype=jnp.float32):
    return lax.dot_general(a, b, (((1,), (0,)), ((), ())),
                           preferred_element_type=out_dtype)


def _mm_body(n_real_wrows, ef_ref, ng_ref, r_ref, s_ref, wr_ref, br_ref,
             out_ref):
    ef = ef_ref[...]
    ng = ng_ref[...].astype(jnp.bfloat16)
    z = (_dot(ef, r_ref[...]) * _dot(ng, s_ref[...])).astype(jnp.bfloat16)
    m = _dot(z, wr_ref[...]) + _dot(ng, br_ref[...])
    row0 = pl.program_id(0) * TILE_W
    rid = row0 + lax.broadcasted_iota(jnp.int32, (TILE_W, 128), 0)
    out_ref[...] = jnp.where(rid < n_real_wrows, m, 0.0)


def _add_body(p_ref, o_ref):
    o_ref[...] = p_ref[0] + p_ref[1]


def _pad_body(n_real_rows, n_pad_rows, s_ref, t_ref, sp_ref, tp_ref):
    zero_tail = jnp.zeros((n_pad_rows, ROW), jnp.int32)
    sp_ref[pl.ds(0, n_real_rows), :] = s_ref[...]
    sp_ref[pl.ds(n_real_rows, n_pad_rows), :] = zero_tail
    tp_ref[pl.ds(0, n_real_rows), :] = t_ref[...]
    tp_ref[pl.ds(n_real_rows, n_pad_rows), :] = zero_tail


def kernel(node_features, edge_features, edge_sources, edge_targets, hidden,
           initial, W, b):
    E = edge_features.shape[0]
    granule = NW * CHUNK_ROWS * ROW
    ep = ((E + granule - 1) // granule) * granule

    n_real_rows = E // ROW
    n_pad_rows = ep // ROW - n_real_rows
    src_r = edge_sources.reshape(n_real_rows, ROW)
    tgt_r = edge_targets.reshape(n_real_rows, ROW)
    src_p, tgt_p = pl.pallas_call(
        functools.partial(_pad_body, n_real_rows, n_pad_rows),
        out_shape=(jax.ShapeDtypeStruct((ep // ROW, ROW), jnp.int32),
                   jax.ShapeDtypeStruct((ep // ROW, ROW), jnp.int32)),
    )(src_r, tgt_r)

    neigh = _make_gather(ep)(hidden, src_p)

    Wr = W.reshape(D_EDGE, MSG, HIDDEN).transpose(0, 2, 1).reshape(
        D_EDGE * HIDDEN, MSG).astype(jnp.bfloat16)
    Br = b.reshape(MSG, HIDDEN).T.astype(jnp.bfloat16)
    R = jnp.repeat(jnp.eye(D_EDGE, dtype=jnp.bfloat16), HIDDEN, axis=1)
    S = jnp.tile(jnp.eye(HIDDEN, dtype=jnp.bfloat16), (1, D_EDGE))
    eye8 = jnp.eye(PACK, dtype=jnp.bfloat16)
    Rb = jnp.kron(eye8, R)
    Sb = jnp.kron(eye8, S)
    Wrb = jnp.kron(eye8, Wr)
    Brb = jnp.kron(eye8, Br)

    ef_w = edge_features.astype(jnp.bfloat16).reshape(E // PACK, 128)
    neigh_w = neigh.reshape(ep // PACK, 128)
    n_tiles = ep // PACK // TILE_W
    n_real_wrows = E // PACK
    clamp = n_real_wrows // TILE_W - 1
    messages = pl.pallas_call(
        functools.partial(_mm_body, n_real_wrows),
        grid=(n_tiles,),
        in_specs=[
            pl.BlockSpec((TILE_W, 128), lambda i: (jnp.minimum(i, clamp), 0)),
            pl.BlockSpec((TILE_W, 128), lambda i: (i, 0)),
            pl.BlockSpec((128, PACK * D_EDGE * HIDDEN), lambda i: (0, 0)),
            pl.BlockSpec((128, PACK * D_EDGE * HIDDEN), lambda i: (0, 0)),
            pl.BlockSpec((PACK * D_EDGE * HIDDEN, 128), lambda i: (0, 0)),
            pl.BlockSpec((128, 128), lambda i: (0, 0)),
        ],
        out_specs=pl.BlockSpec((TILE_W, 128), lambda i: (i, 0)),
        out_shape=jax.ShapeDtypeStruct((ep // PACK, 128), jnp.float32),
    )(ef_w, neigh_w, Rb, Sb, Wrb, Brb)

    zeros = jnp.zeros((N_NODES, MSG), jnp.float32)
    partials = _make_scatter(ep)(messages.reshape(ep, MSG), tgt_p, zeros)

    out = pl.pallas_call(
        _add_body,
        out_shape=jax.ShapeDtypeStruct((N_NODES, MSG), jnp.float32),
    )(partials)
    return out


# speedup vs baseline: 1.0823x; 1.0596x over previous
import functools

import jax
import jax.numpy as jnp
from jax import lax
from jax.experimental import pallas as pl
from jax.experimental.pallas import tpu as pltpu
from jax.experimental.pallas import tpu_sc as plsc

N_NODES = 10000
D_EDGE = 16
HIDDEN = 16
MSG = 16

_SC_PARAMS = pltpu.CompilerParams(use_tc_tiling_on_sc=False)

NC, NS = 2, 16
NW = NC * NS
ROW = 128
CHUNK_ROWS = 40
CHUNK_EDGES = CHUNK_ROWS * ROW
PACK = 128 // HIDDEN
TILE_W = 1024


def _make_gather(ep):
    rows_per_worker = (ep // ROW) // NW
    n_chunks = rows_per_worker // CHUNK_ROWS
    mesh = plsc.VectorSubcoreMesh(core_axis_name="c", subcore_axis_name="s")

    @functools.partial(
        pl.kernel,
        out_type=jax.ShapeDtypeStruct((ep, HIDDEN), jnp.float32),
        mesh=mesh,
        scratch_types=[
            pltpu.VMEM((CHUNK_ROWS, ROW), jnp.int32),
            pltpu.VMEM((CHUNK_EDGES, HIDDEN), jnp.float32),
            pltpu.VMEM_SHARED((N_NODES, HIDDEN), jnp.float32),
            pltpu.SemaphoreType.DMA,
        ],
        compiler_params=_SC_PARAMS,
    )
    def gather(hidden_hbm, src_hbm, neigh_hbm, idx_v, rows_v, hid_sh, sem):
        cid = lax.axis_index("c")
        sid = lax.axis_index("s")
        wid = sid * NC + cid

        @pl.when(sid == 0)
        def _():
            pltpu.sync_copy(hidden_hbm, hid_sh)

        plsc.subcore_barrier()

        @pl.loop(0, n_chunks)
        def _(ci):
            row0 = wid * rows_per_worker + ci * CHUNK_ROWS

            pltpu.sync_copy(src_hbm.at[pl.ds(row0, CHUNK_ROWS)], idx_v)

            @pl.loop(0, CHUNK_ROWS)
            def _(j):
                pltpu.async_copy(
                    hid_sh.at[idx_v.at[j]],
                    rows_v.at[pl.ds(j * ROW, ROW)],
                    sem,
                )

            pltpu.make_async_copy(
                neigh_hbm.at[pl.ds(0, CHUNK_EDGES)], rows_v, sem
            ).wait()

            pltpu.sync_copy(rows_v, neigh_hbm.at[pl.ds(row0 * ROW, CHUNK_EDGES)])

    return gather


def _make_scatter(ep):
    rows_per_worker = (ep // ROW) // NW
    n_chunks = rows_per_worker // CHUNK_ROWS
    mesh = plsc.VectorSubcoreMesh(core_axis_name="c", subcore_axis_name="s")

    @functools.partial(
        pl.kernel,
        out_type=jax.ShapeDtypeStruct((NC, N_NODES, MSG), jnp.float32),
        mesh=mesh,
        scratch_types=[
            pltpu.VMEM((CHUNK_ROWS, ROW), jnp.int32),
            pltpu.VMEM((CHUNK_EDGES, MSG), jnp.float32),
            pltpu.VMEM_SHARED((N_NODES, MSG), jnp.float32),
            pltpu.SemaphoreType.DMA,
        ],
        compiler_params=_SC_PARAMS,
    )
    def scatter(msg_hbm, tgt_hbm, zero_hbm, out_hbm, idx_v, rows_v, acc_sh, sem):
        cid = lax.axis_index("c")
        sid = lax.axis_index("s")

        @pl.when(sid == 0)
        def _():
            pltpu.sync_copy(zero_hbm, acc_sh)

        plsc.subcore_barrier()

        @pl.loop(0, n_chunks)
        def _(ci):
            row0 = (cid * NS + sid) * rows_per_worker + ci * CHUNK_ROWS

            pltpu.sync_copy(tgt_hbm.at[pl.ds(row0, CHUNK_ROWS)], idx_v)
            pltpu.sync_copy(msg_hbm.at[pl.ds(row0 * ROW, CHUNK_EDGES)], rows_v)

            @pl.loop(0, CHUNK_ROWS)
            def _(j):
                pltpu.sync_copy(
                    rows_v.at[pl.ds(j * ROW, ROW)],
                    acc_sh.at[idx_v.at[j]],
                    add=True,
                )

        plsc.subcore_barrier()

        @pl.when(sid == 0)
        def _():
            pltpu.sync_copy(acc_sh, out_hbm.at[cid])

    return scatter


def _dot(a, b, out_dtype=jnp.float32):
    return lax.dot_general(a, b, (((1,), (0,)), ((), ())),
                           preferred_element_type=out_dtype)


def _mm_body(n_real_wrows, ef_ref, ng_ref, r_ref, s_ref, wr_ref, br_ref,
             out_ref):
    ef = ef_ref[...]
    ng = ng_ref[...].astype(jnp.bfloat16)
    z = (_dot(ef, r_ref[...]).astype(jnp.bfloat16) *
         _dot(ng, s_ref[...]).astype(jnp.bfloat16))
    m = _dot(z, wr_ref[...]) + _dot(ng, br_ref[...])
    row0 = pl.program_id(0) * TILE_W
    rid = row0 + lax.broadcasted_iota(jnp.int32, (TILE_W, 128), 0)
    out_ref[...] = jnp.where(rid < n_real_wrows, m, 0.0)


def _add_body(p_ref, o_ref):
    o_ref[...] = p_ref[0] + p_ref[1]


def _pad_body(n_real_rows, n_pad_rows, s_ref, t_ref, sp_ref, tp_ref):
    zero_tail = jnp.zeros((n_pad_rows, ROW), jnp.int32)
    sp_ref[pl.ds(0, n_real_rows), :] = s_ref[...]
    sp_ref[pl.ds(n_real_rows, n_pad_rows), :] = zero_tail
    tp_ref[pl.ds(0, n_real_rows), :] = t_ref[...]
    tp_ref[pl.ds(n_real_rows, n_pad_rows), :] = zero_tail


def kernel(node_features, edge_features, edge_sources, edge_targets, hidden,
           initial, W, b):
    E = edge_features.shape[0]
    granule = NW * CHUNK_ROWS * ROW
    ep = ((E + granule - 1) // granule) * granule

    n_real_rows = E // ROW
    n_pad_rows = ep // ROW - n_real_rows
    src_r = edge_sources.reshape(n_real_rows, ROW)
    tgt_r = edge_targets.reshape(n_real_rows, ROW)
    src_p, tgt_p = pl.pallas_call(
        functools.partial(_pad_body, n_real_rows, n_pad_rows),
        out_shape=(jax.ShapeDtypeStruct((ep // ROW, ROW), jnp.int32),
                   jax.ShapeDtypeStruct((ep // ROW, ROW), jnp.int32)),
    )(src_r, tgt_r)

    neigh = _make_gather(ep)(hidden, src_p)

    Wr = W.reshape(D_EDGE, MSG, HIDDEN).transpose(0, 2, 1).reshape(
        D_EDGE * HIDDEN, MSG).astype(jnp.bfloat16)
    Br = b.reshape(MSG, HIDDEN).T.astype(jnp.bfloat16)
    R = jnp.repeat(jnp.eye(D_EDGE, dtype=jnp.bfloat16), HIDDEN, axis=1)
    S = jnp.tile(jnp.eye(HIDDEN, dtype=jnp.bfloat16), (1, D_EDGE))
    eye8 = jnp.eye(PACK, dtype=jnp.bfloat16)
    Rb = jnp.kron(eye8, R)
    Sb = jnp.kron(eye8, S)
    Wrb = jnp.kron(eye8, Wr)
    Brb = jnp.kron(eye8, Br)

    n_real_wrows = E // PACK
    ef_w = jnp.concatenate([
        edge_features.astype(jnp.bfloat16).reshape(n_real_wrows, 128),
        jnp.zeros((ep // PACK - n_real_wrows, 128), jnp.bfloat16)])
    neigh_w = neigh.reshape(ep // PACK, 128)
    n_tiles = ep // PACK // TILE_W
    messages = pl.pallas_call(
        functools.partial(_mm_body, n_real_wrows),
        grid=(n_tiles,),
        in_specs=[
            pl.BlockSpec((TILE_W, 128), lambda i: (i, 0)),
            pl.BlockSpec((TILE_W, 128), lambda i: (i, 0)),
            pl.BlockSpec((128, PACK * D_EDGE * HIDDEN), lambda i: (0, 0)),
            pl.BlockSpec((128, PACK * D_EDGE * HIDDEN), lambda i: (0, 0)),
            pl.BlockSpec((PACK * D_EDGE * HIDDEN, 128), lambda i: (0, 0)),
            pl.BlockSpec((128, 128), lambda i: (0, 0)),
        ],
        out_specs=pl.BlockSpec((TILE_W, 128), lambda i: (i, 0)),
        out_shape=jax.ShapeDtypeStruct((ep // PACK, 128), jnp.float32),
    )(ef_w, neigh_w, Rb, Sb, Wrb, Brb)

    zeros = jnp.zeros((N_NODES, MSG), jnp.float32)
    partials = _make_scatter(ep)(messages.reshape(ep, MSG), tgt_p, zeros)

    out = pl.pallas_call(
        _add_body,
        out_shape=jax.ShapeDtypeStruct((N_NODES, MSG), jnp.float32),
    )(partials)
    return out
